# Initial kernel scaffold; baseline (speedup 1.0000x reference)
import jax
import jax.numpy as jnp
from jax.experimental import pallas as pl


def kernel(x, self_rpos, field, sa_qw, sa_kw, sa_vw, sa_projw, ca_qw, ca_kw, ca_vw, ca_gw, ca_projw, n0w, n0b, n1w, n1b, n2w, n2b, fc1w, fc1b, dww, dwb, fc2w, fc2b, field_scale):
    raise NotImplementedError("write your pallas kernel here")


# speedup vs baseline: 25.5978x; 25.5978x over previous
import functools

import jax
import jax.numpy as jnp
from jax import lax
from jax.experimental import pallas as pl
from jax.experimental.pallas import tpu as pltpu
from jax.experimental.pallas import tpu_sc as plsc

B, H, W, DIM = 2, 128, 128, 96
NH, HD, R = 8, 32, 1
FD = 2
A = (2 * R + 2) * (2 * R + 2)
DIM_S = DIM + FD + NH * 2
EMB_S = DIM_S + 1
DIM_C = DIM + FD
AD = NH * HD
HID = DIM * 2
SCALE = HD ** (-0.5)
N = H * W
NC, NS, LANES = 2, 16, 16
NW = NC * NS
M = B * N * NH
CH = 8
IPW = M // NW

_CORNERS = ((0, 0), (1, 0), (0, 1), (1, 1))


def _make_attn_core():
    mesh = plsc.VectorSubcoreMesh(
        core_axis_name="c", subcore_axis_name="s",
        num_cores=NC, num_subcores=NS)

    @functools.partial(
        pl.kernel,
        out_type=[jax.ShapeDtypeStruct((M, HD), jnp.float32),
                  jax.ShapeDtypeStruct((M, A), jnp.float32)],
        mesh=mesh,
        compiler_params=pltpu.CompilerParams(
            use_tc_tiling_on_sc=False, needs_layout_passes=False),
        scratch_types=[
            pltpu.VMEM((CH, HD), jnp.float32),
            pltpu.VMEM((CH * 4,), jnp.float32),
            pltpu.VMEM((CH * A,), jnp.int32),
            pltpu.VMEM((CH * A, 2 * HD), jnp.float32),
            pltpu.VMEM((CH, HD), jnp.float32),
            pltpu.VMEM((CH, A), jnp.float32),
            pltpu.SemaphoreType.DMA,
            pltpu.SemaphoreType.DMA,
        ])
    def attn_core(q_hbm, kv_hbm, idx_hbm, bw_hbm, o_hbm, p_hbm,
                  q_v, bw_v, idx_v, rows_v, o_v, p_v, isem, gsem):
        cid = lax.axis_index("c")
        sid = lax.axis_index("s")
        wid = cid * NS + sid

        def chunk_body(c, carry):
            lane = lax.iota(jnp.int32, 16)
            dyv = lane // 4
            dxv = lane - dyv * 4
            msks = [((dyv >= dyc) & (dyv <= dyc + 2)
                     & (dxv >= dxc) & (dxv <= dxc + 2))
                    for (dxc, dyc) in _CORNERS]
            dcols = [jnp.full((LANES,), d, jnp.int32) for d in range(HD)]
            g0 = wid * IPW + c * CH
            cp_q = pltpu.async_copy(q_hbm.at[pl.ds(g0, CH)], q_v, isem)
            cp_b = pltpu.async_copy(bw_hbm.at[pl.ds(g0 * 4, CH * 4)],
                                    bw_v, isem)
            cp_i = pltpu.async_copy(idx_hbm.at[pl.ds(g0 * A, CH * A)],
                                    idx_v, isem)
            cp_q.wait()
            cp_b.wait()
            cp_i.wait()
            pltpu.async_copy(kv_hbm.at[idx_v], rows_v, gsem).wait()
            bw_lo = bw_v[pl.ds(0, LANES)]
            bw_hi = bw_v[pl.ds(LANES, LANES)]
            for i in range(CH):
                row0 = i * A
                rlane = row0 + lane
                qa = q_v[i, pl.ds(0, LANES)]
                qb = q_v[i, pl.ds(LANES, LANES)]
                acc = jnp.zeros((LANES,), jnp.float32)
                for d in range(HD):
                    kcol = plsc.load_gather(rows_v, [rlane, dcols[d]])
                    qd = qa[d] if d < LANES else qb[d - LANES]
                    acc = acc + kcol * qd
                pv = jnp.zeros((LANES,), jnp.float32)
                for ci in range(4):
                    m = msks[ci]
                    wv = jnp.where(m, acc, -1e30)
                    mx = jnp.max(wv)
                    e = jnp.where(m, jnp.exp(acc - mx), 0.0)
                    j = i * 4 + ci
                    bwc = bw_lo[j] if j < LANES else bw_hi[j - LANES]
                    pv = pv + (e * bwc) / jnp.sum(e)
                p_v[i] = pv
                acc0 = jnp.zeros((LANES,), jnp.float32)
                acc1 = jnp.zeros((LANES,), jnp.float32)
                for a in range(A):
                    pa = pv[a]
                    acc0 = acc0 + rows_v[row0 + a, pl.ds(HD, LANES)] * pa
                    acc1 = acc1 + rows_v[row0 + a,
                                         pl.ds(HD + LANES, LANES)] * pa
                o_v[i, pl.ds(0, LANES)] = acc0
                o_v[i, pl.ds(LANES, LANES)] = acc1
            pltpu.sync_copy(o_v, o_hbm.at[pl.ds(g0, CH)])
            pltpu.sync_copy(p_v, p_hbm.at[pl.ds(g0, CH)])
            return carry

        lax.fori_loop(0, IPW // CH, chunk_body, 0)

    return attn_core


_ATTN_CORE = _make_attn_core()


def _init_coords():
    gx, gy = jnp.meshgrid(jnp.arange(W, dtype=jnp.float32),
                          jnp.arange(H, dtype=jnp.float32), indexing='xy')
    return jnp.stack([gx, gy], axis=-1)[None]


def _layernorm(x, w, b, eps=1e-5):
    m = jnp.mean(x, -1, keepdims=True)
    v = jnp.var(x, -1, keepdims=True)
    return (x - m) / jnp.sqrt(v + eps) * w + b


def _window_meta(max_offset):
    ox = jnp.clip(max_offset[..., 0:1], float(R), W - 1 - R - 0.001)
    oy = jnp.clip(max_offset[..., 1:2], float(R), H - 1 - R - 0.001)
    mo = jnp.concatenate([ox, oy], -1)
    mf = jnp.floor(mo)
    f = mo - mf
    fx, fy = f[..., 0], f[..., 1]
    bw = jnp.stack([(1 - fx) * (1 - fy), fx * (1 - fy),
                    (1 - fx) * fy, fx * fy], -1)
    m_id = mf.astype(jnp.int32)
    base = m_id[..., 1] * W + m_id[..., 0]
    dy, dx = jnp.meshgrid(jnp.arange(-R, R + 2), jnp.arange(-R, R + 2),
                          indexing='ij')
    off = (dy * W + dx).reshape(-1).astype(jnp.int32)
    pos = jnp.clip(base[..., None] + off, 0, N - 1)
    h_arr = jnp.arange(NH, dtype=jnp.int32)[None, None, :, None]
    b_arr = jnp.arange(B, dtype=jnp.int32)[:, None, None, None]
    rows = (pos + b_arr * N) * NH + h_arr
    return rows.reshape(M * A), bw.reshape(M * 4)


def _attn_sc(q, k, v, max_offset):
    idx_sc, bw_sc = _window_meta(max_offset)
    q_sc = q.reshape(M, HD)
    kv_sc = jnp.concatenate([k, v], -1).reshape(M, 2 * HD)
    o, p = _ATTN_CORE(q_sc, kv_sc, idx_sc, bw_sc)
    return o.reshape(B, N, AD), p.reshape(B, N, NH, A)


def _bilinear_sample(img, coords):
    b, h, w, c = img.shape
    x = jnp.clip(coords[..., 0], 0.0, w - 1.0)
    y = jnp.clip(coords[..., 1], 0.0, h - 1.0)
    x0 = jnp.clip(jnp.floor(x).astype(jnp.int32), 0, w - 2)
    y0 = jnp.clip(jnp.floor(y).astype(jnp.int32), 0, h - 2)
    wx = (x - x0)[..., None]
    wy = (y - y0)[..., None]
    flat = img.reshape(b, h * w, c)

    def g(yy, xx):
        i = (yy * w + xx).reshape(b, h * w)
        return jnp.take_along_axis(flat, i[..., None], axis=1).reshape(b, h, w, c)

    v00 = g(y0, x0)
    v01 = g(y0, x0 + 1)
    v10 = g(y0 + 1, x0)
    v11 = g(y0 + 1, x0 + 1)
    return (v00 * (1 - wx) * (1 - wy) + v01 * wx * (1 - wy)
            + v10 * (1 - wx) * wy + v11 * wx * wy)


def _consistency_mask(field, a_thr=2.0):
    offset = field + _init_coords()
    half = B // 2
    field_tgt = jnp.concatenate([field[half:], field[:half]], axis=0)
    sampled = _bilinear_sample(field_tgt, offset)
    diff = jnp.abs(field + sampled).sum(-1, keepdims=True)
    return (diff < a_thr).astype(field.dtype)


def kernel(x, self_rpos, field, sa_qw, sa_kw, sa_vw, sa_projw, ca_qw, ca_kw,
           ca_vw, ca_gw, ca_projw, n0w, n0b, n1w, n1b, n2w, n2b, fc1w, fc1b,
           dww, dwb, fc2w, fc2b, field_scale):
    noc_mask = _consistency_mask(lax.stop_gradient(field))
    xx = jnp.concatenate([x, field * field_scale, self_rpos], -1)
    coords0 = jnp.tile(_init_coords(), (1, 1, 1, NH))
    self_offset = (self_rpos + coords0).reshape(B, N, NH, 2)

    xs = _layernorm(xx, n0w, n0b).reshape(B, N, DIM_S)
    xs = jnp.concatenate([xs, noc_mask.reshape(B, N, 1)], -1)
    q = (xs @ (sa_qw * SCALE)).reshape(B, N, NH, HD)
    k = (xs @ sa_kw).reshape(B, N, NH, HD)
    v = (xs @ sa_vw).reshape(B, N, NH, HD)
    out, _ = _attn_sc(q, k, v, self_offset)
    out = (out @ sa_projw).reshape(B, H, W, DIM_S)
    xx = xx + out

    self_rpos_out = xx[..., -(NH * 2):]
    xx = xx[..., :-(NH * 2)]
    xx = xx.at[..., -1].set(0.0)
    field1 = xx[..., -FD:] / field_scale
    offset = (jnp.tile(field1, (1, 1, 1, NH)) + coords0).reshape(B, N, NH, 2)

    xc = _layernorm(xx, n1w, n1b).reshape(B, N, DIM_C)
    half = B // 2
    tgt = jnp.concatenate([xc[half:], xc[:half]], axis=0)
    g = jax.nn.silu(xc @ ca_gw)
    q = (xc @ (ca_qw * SCALE)).reshape(B, N, NH, HD)
    k = (tgt @ ca_kw).reshape(B, N, NH, HD)
    v = (tgt @ ca_vw).reshape(B, N, NH, HD)
    out, probs = _attn_sc(q, k, v, offset)
    out = jnp.concatenate([g * out, probs.reshape(B, N, NH * A)], -1)
    out = (out @ ca_projw).reshape(B, H, W, DIM_C)
    xx = xx + out

    xx = xx.at[..., -1].set(0.0)
    field2 = xx[..., -FD:] / field_scale
    xx = xx[..., :-FD]

    hh = _layernorm(xx, n2w, n2b) @ fc1w + fc1b
    a, gg = jnp.split(hh, 2, axis=-1)
    a = lax.conv_general_dilated(
        a, dww, (1, 1), 'SAME',
        dimension_numbers=('NHWC', 'HWIO', 'NHWC'),
        feature_group_count=HID) + dwb
    xx = xx + (jax.nn.gelu(a) * gg) @ fc2w + fc2b
    return (xx, self_rpos_out, field2, field1, field2)


# speedup vs baseline: 33.4618x; 1.3072x over previous
import functools

import jax
import jax.numpy as jnp
from jax import lax
from jax.experimental import pallas as pl
from jax.experimental.pallas import tpu as pltpu
from jax.experimental.pallas import tpu_sc as plsc

B, H, W, DIM = 2, 128, 128, 96
NH, HD, R = 8, 32, 1
FD = 2
A = (2 * R + 2) * (2 * R + 2)
DIM_S = DIM + FD + NH * 2
EMB_S = DIM_S + 1
DIM_C = DIM + FD
AD = NH * HD
HID = DIM * 2
SCALE = HD ** (-0.5)
N = H * W
NC, NS, LANES = 2, 16, 16
NW = NC * NS
M = B * N * NH
CH = 32
NG = (CH * A) // 128
IPW = M // NW

_CORNERS = ((0, 0), (1, 0), (0, 1), (1, 1))


def _make_attn_core():
    mesh = plsc.VectorSubcoreMesh(
        core_axis_name="c", subcore_axis_name="s",
        num_cores=NC, num_subcores=NS)

    @functools.partial(
        pl.kernel,
        out_type=[jax.ShapeDtypeStruct((M, HD), jnp.float32),
                  jax.ShapeDtypeStruct((M, A), jnp.float32)],
        mesh=mesh,
        compiler_params=pltpu.CompilerParams(
            use_tc_tiling_on_sc=False, needs_layout_passes=False),
        scratch_types=[
            pltpu.VMEM((CH, HD), jnp.float32),
            pltpu.VMEM((CH * 4 + LANES,), jnp.float32),
            pltpu.VMEM((CH * A,), jnp.int32),
            pltpu.VMEM((CH * A, 2 * HD), jnp.float32),
            pltpu.VMEM((CH, HD), jnp.float32),
            pltpu.VMEM((CH, A), jnp.float32),
            pltpu.SemaphoreType.DMA,
            pltpu.SemaphoreType.DMA,
        ])
    def attn_core(q_hbm, kv_hbm, idx_hbm, bw_hbm, o_hbm, p_hbm,
                  q_v, bw_v, idx_v, rows_v, o_v, p_v, isem, gsem):
        cid = lax.axis_index("c")
        sid = lax.axis_index("s")
        wid = cid * NS + sid

        def item_body(i, carry):
            lane = lax.iota(jnp.int32, 16)
            dyv = lane // 4
            dxv = lane - dyv * 4
            msks = [((dyv >= dyc) & (dyv <= dyc + 2)
                     & (dxv >= dxc) & (dxv <= dxc + 2))
                    for (dxc, dyc) in _CORNERS]
            row0 = i * A
            rlane = row0 + lane
            qa = q_v[i, pl.ds(0, LANES)]
            qb = q_v[i, pl.ds(LANES, LANES)]
            acc = jnp.zeros((LANES,), jnp.float32)
            for d in range(HD):
                dcol = jnp.full((LANES,), d, jnp.int32)
                kcol = plsc.load_gather(rows_v, [rlane, dcol])
                qd = qa[d] if d < LANES else qb[d - LANES]
                acc = acc + kcol * qd
            bwv = bw_v[pl.ds(i * 4, LANES)]
            pv = jnp.zeros((LANES,), jnp.float32)
            for ci in range(4):
                m = msks[ci]
                wv = jnp.where(m, acc, -1e30)
                mx = jnp.max(wv)
                e = jnp.where(m, jnp.exp(acc - mx), 0.0)
                pv = pv + (e * bwv[ci]) / jnp.sum(e)
            p_v[i] = pv
            acc0 = jnp.zeros((LANES,), jnp.float32)
            acc1 = jnp.zeros((LANES,), jnp.float32)
            for a in range(A):
                pa = pv[a]
                acc0 = acc0 + rows_v[row0 + a, pl.ds(HD, LANES)] * pa
                acc1 = acc1 + rows_v[row0 + a, pl.ds(HD + LANES, LANES)] * pa
            o_v[i, pl.ds(0, LANES)] = acc0
            o_v[i, pl.ds(LANES, LANES)] = acc1
            return carry

        def chunk_body(c, carry):
            g0 = wid * IPW + c * CH
            cp_q = pltpu.async_copy(q_hbm.at[pl.ds(g0, CH)], q_v, isem)
            cp_b = pltpu.async_copy(bw_hbm.at[pl.ds(g0 * 4, CH * 4)],
                                    bw_v.at[pl.ds(0, CH * 4)], isem)
            cp_i = pltpu.async_copy(idx_hbm.at[pl.ds(g0 * A, CH * A)],
                                    idx_v, isem)
            cp_q.wait()
            cp_b.wait()
            cp_i.wait()
            gs = [pltpu.async_copy(
                      kv_hbm.at[idx_v.at[pl.ds(j * 128, 128)]],
                      rows_v.at[pl.ds(j * 128, 128)], gsem)
                  for j in range(NG)]
            for cp in gs:
                cp.wait()
            lax.fori_loop(0, CH, item_body, 0)
            pltpu.sync_copy(o_v, o_hbm.at[pl.ds(g0, CH)])
            pltpu.sync_copy(p_v, p_hbm.at[pl.ds(g0, CH)])
            return carry

        lax.fori_loop(0, IPW // CH, chunk_body, 0)

    return attn_core


_ATTN_CORE = _make_attn_core()


def _init_coords():
    gx, gy = jnp.meshgrid(jnp.arange(W, dtype=jnp.float32),
                          jnp.arange(H, dtype=jnp.float32), indexing='xy')
    return jnp.stack([gx, gy], axis=-1)[None]


def _layernorm(x, w, b, eps=1e-5):
    m = jnp.mean(x, -1, keepdims=True)
    v = jnp.var(x, -1, keepdims=True)
    return (x - m) / jnp.sqrt(v + eps) * w + b


def _window_meta(max_offset):
    ox = jnp.clip(max_offset[..., 0:1], float(R), W - 1 - R - 0.001)
    oy = jnp.clip(max_offset[..., 1:2], float(R), H - 1 - R - 0.001)
    mo = jnp.concatenate([ox, oy], -1)
    mf = jnp.floor(mo)
    f = mo - mf
    fx, fy = f[..., 0], f[..., 1]
    bw = jnp.stack([(1 - fx) * (1 - fy), fx * (1 - fy),
                    (1 - fx) * fy, fx * fy], -1)
    m_id = mf.astype(jnp.int32)
    base = m_id[..., 1] * W + m_id[..., 0]
    dy, dx = jnp.meshgrid(jnp.arange(-R, R + 2), jnp.arange(-R, R + 2),
                          indexing='ij')
    off = (dy * W + dx).reshape(-1).astype(jnp.int32)
    pos = jnp.clip(base[..., None] + off, 0, N - 1)
    h_arr = jnp.arange(NH, dtype=jnp.int32)[None, None, :, None]
    b_arr = jnp.arange(B, dtype=jnp.int32)[:, None, None, None]
    rows = (pos + b_arr * N) * NH + h_arr
    return rows.reshape(M * A), bw.reshape(M * 4)


def _attn_sc(q, k, v, max_offset):
    idx_sc, bw_sc = _window_meta(max_offset)
    q_sc = q.reshape(M, HD)
    kv_sc = jnp.concatenate([k, v], -1).reshape(M, 2 * HD)
    o, p = _ATTN_CORE(q_sc, kv_sc, idx_sc, bw_sc)
    return o.reshape(B, N, AD), p.reshape(B, N, NH, A)


def _bilinear_sample(img, coords):
    b, h, w, c = img.shape
    x = jnp.clip(coords[..., 0], 0.0, w - 1.0)
    y = jnp.clip(coords[..., 1], 0.0, h - 1.0)
    x0 = jnp.clip(jnp.floor(x).astype(jnp.int32), 0, w - 2)
    y0 = jnp.clip(jnp.floor(y).astype(jnp.int32), 0, h - 2)
    wx = (x - x0)[..., None]
    wy = (y - y0)[..., None]
    flat = img.reshape(b, h * w, c)

    def g(yy, xx):
        i = (yy * w + xx).reshape(b, h * w)
        return jnp.take_along_axis(flat, i[..., None], axis=1).reshape(b, h, w, c)

    v00 = g(y0, x0)
    v01 = g(y0, x0 + 1)
    v10 = g(y0 + 1, x0)
    v11 = g(y0 + 1, x0 + 1)
    return (v00 * (1 - wx) * (1 - wy) + v01 * wx * (1 - wy)
            + v10 * (1 - wx) * wy + v11 * wx * wy)


def _consistency_mask(field, a_thr=2.0):
    offset = field + _init_coords()
    half = B // 2
    field_tgt = jnp.concatenate([field[half:], field[:half]], axis=0)
    sampled = _bilinear_sample(field_tgt, offset)
    diff = jnp.abs(field + sampled).sum(-1, keepdims=True)
    return (diff < a_thr).astype(field.dtype)


def kernel(x, self_rpos, field, sa_qw, sa_kw, sa_vw, sa_projw, ca_qw, ca_kw,
           ca_vw, ca_gw, ca_projw, n0w, n0b, n1w, n1b, n2w, n2b, fc1w, fc1b,
           dww, dwb, fc2w, fc2b, field_scale):
    noc_mask = _consistency_mask(lax.stop_gradient(field))
    xx = jnp.concatenate([x, field * field_scale, self_rpos], -1)
    coords0 = jnp.tile(_init_coords(), (1, 1, 1, NH))
    self_offset = (self_rpos + coords0).reshape(B, N, NH, 2)

    xs = _layernorm(xx, n0w, n0b).reshape(B, N, DIM_S)
    xs = jnp.concatenate([xs, noc_mask.reshape(B, N, 1)], -1)
    q = (xs @ (sa_qw * SCALE)).reshape(B, N, NH, HD)
    k = (xs @ sa_kw).reshape(B, N, NH, HD)
    v = (xs @ sa_vw).reshape(B, N, NH, HD)
    out, _ = _attn_sc(q, k, v, self_offset)
    out = (out @ sa_projw).reshape(B, H, W, DIM_S)
    xx = xx + out

    self_rpos_out = xx[..., -(NH * 2):]
    xx = xx[..., :-(NH * 2)]
    xx = xx.at[..., -1].set(0.0)
    field1 = xx[..., -FD:] / field_scale
    offset = (jnp.tile(field1, (1, 1, 1, NH)) + coords0).reshape(B, N, NH, 2)

    xc = _layernorm(xx, n1w, n1b).reshape(B, N, DIM_C)
    half = B // 2
    tgt = jnp.concatenate([xc[half:], xc[:half]], axis=0)
    g = jax.nn.silu(xc @ ca_gw)
    q = (xc @ (ca_qw * SCALE)).reshape(B, N, NH, HD)
    k = (tgt @ ca_kw).reshape(B, N, NH, HD)
    v = (tgt @ ca_vw).reshape(B, N, NH, HD)
    out, probs = _attn_sc(q, k, v, offset)
    out = jnp.concatenate([g * out, probs.reshape(B, N, NH * A)], -1)
    out = (out @ ca_projw).reshape(B, H, W, DIM_C)
    xx = xx + out

    xx = xx.at[..., -1].set(0.0)
    field2 = xx[..., -FD:] / field_scale
    xx = xx[..., :-FD]

    hh = _layernorm(xx, n2w, n2b) @ fc1w + fc1b
    a, gg = jnp.split(hh, 2, axis=-1)
    a = lax.conv_general_dilated(
        a, dww, (1, 1), 'SAME',
        dimension_numbers=('NHWC', 'HWIO', 'NHWC'),
        feature_group_count=HID) + dwb
    xx = xx + (jax.nn.gelu(a) * gg) @ fc2w + fc2b
    return (xx, self_rpos_out, field2, field1, field2)


# speedup vs baseline: 56.7324x; 1.6954x over previous
import functools

import jax
import jax.numpy as jnp
from jax import lax
from jax.experimental import pallas as pl
from jax.experimental.pallas import tpu as pltpu
from jax.experimental.pallas import tpu_sc as plsc

B, H, W, DIM = 2, 128, 128, 96
NH, HD, R = 8, 32, 1
FD = 2
A = (2 * R + 2) * (2 * R + 2)
DIM_S = DIM + FD + NH * 2
EMB_S = DIM_S + 1
DIM_C = DIM + FD
AD = NH * HD
HID = DIM * 2
SCALE = HD ** (-0.5)
N = H * W
NC, NS, LANES = 2, 16, 16
NW = NC * NS
M = B * N * NH
CH = 32
NG = (CH * A) // 128
IPW = M // NW

_CORNERS = ((0, 0), (1, 0), (0, 1), (1, 1))


def _make_attn_core():
    mesh = plsc.VectorSubcoreMesh(
        core_axis_name="c", subcore_axis_name="s",
        num_cores=NC, num_subcores=NS)

    @functools.partial(
        pl.kernel,
        out_type=[jax.ShapeDtypeStruct((M, HD), jnp.float32),
                  jax.ShapeDtypeStruct((M, A), jnp.float32)],
        mesh=mesh,
        compiler_params=pltpu.CompilerParams(
            use_tc_tiling_on_sc=False, needs_layout_passes=False),
        scratch_types=[
            pltpu.VMEM((CH, HD), jnp.float32),
            pltpu.VMEM((CH, HD), jnp.float32),
            pltpu.VMEM((CH * 4 + LANES,), jnp.float32),
            pltpu.VMEM((CH * 4 + LANES,), jnp.float32),
            pltpu.VMEM((CH * A,), jnp.int32),
            pltpu.VMEM((CH * A,), jnp.int32),
            pltpu.VMEM((CH * A, 2 * HD), jnp.float32),
            pltpu.VMEM((CH * A, 2 * HD), jnp.float32),
            pltpu.VMEM((CH, HD), jnp.float32),
            pltpu.VMEM((CH, A), jnp.float32),
            pltpu.SemaphoreType.DMA,
            pltpu.SemaphoreType.DMA,
        ])
    def attn_core(q_hbm, kv_hbm, idx_hbm, bw_hbm, o_hbm, p_hbm,
                  q_0, q_1, bw_0, bw_1, idx_0, idx_1, rows_0, rows_1,
                  o_v, p_v, isem, gsem):
        cid = lax.axis_index("c")
        sid = lax.axis_index("s")
        wid = cid * NS + sid
        NP = IPW // CH
        bufs = [(q_0, bw_0, idx_0, rows_0), (q_1, bw_1, idx_1, rows_1)]

        def in_copies(c, qv, bwv, idxv):
            g0 = wid * IPW + c * CH
            return [(q_hbm.at[pl.ds(g0, CH)], qv),
                    (bw_hbm.at[pl.ds(g0 * 4, CH * 4)],
                     bwv.at[pl.ds(0, CH * 4)]),
                    (idx_hbm.at[pl.ds(g0 * A, CH * A)], idxv)]

        def issue_in(c, qv, bwv, idxv):
            for s, d in in_copies(c, qv, bwv, idxv):
                pltpu.async_copy(s, d, isem)

        def wait_in(c, qv, bwv, idxv):
            for s, d in in_copies(c, qv, bwv, idxv):
                pltpu.make_async_copy(s, d, isem).wait()

        def issue_gather(idxv, rowsv):
            for j in range(NG):
                pltpu.async_copy(kv_hbm.at[idxv.at[pl.ds(j * 128, 128)]],
                                 rowsv.at[pl.ds(j * 128, 128)], gsem)

        def wait_gather(idxv, rowsv):
            for j in range(NG):
                pltpu.make_async_copy(
                    kv_hbm.at[idxv.at[pl.ds(j * 128, 128)]],
                    rowsv.at[pl.ds(j * 128, 128)], gsem).wait()

        def compute(qv, bwv, rowsv):
            def item_body(i, carry):
                lane = lax.iota(jnp.int32, 16)
                dyv = lane // 4
                dxv = lane - dyv * 4
                msks = [((dyv >= dyc) & (dyv <= dyc + 2)
                         & (dxv >= dxc) & (dxv <= dxc + 2))
                        for (dxc, dyc) in _CORNERS]
                row0 = i * A
                qa = qv[i, pl.ds(0, LANES)]
                qb = qv[i, pl.ds(LANES, LANES)]
                acc = jnp.zeros((LANES,), jnp.float32)
                for a in range(A):
                    k0 = rowsv[row0 + a, pl.ds(0, LANES)]
                    k1 = rowsv[row0 + a, pl.ds(LANES, LANES)]
                    s = jnp.sum(k0 * qa + k1 * qb)
                    acc = jnp.where(lane == a, s, acc)
                bwi = bwv[pl.ds(i * 4, LANES)]
                pv = jnp.zeros((LANES,), jnp.float32)
                for ci in range(4):
                    m = msks[ci]
                    wv = jnp.where(m, acc, -1e30)
                    mx = jnp.max(wv)
                    e = jnp.where(m, jnp.exp(acc - mx), 0.0)
                    pv = pv + (e * bwi[ci]) / jnp.sum(e)
                p_v[i] = pv
                acc0 = jnp.zeros((LANES,), jnp.float32)
                acc1 = jnp.zeros((LANES,), jnp.float32)
                for a in range(A):
                    pa = pv[a]
                    acc0 = acc0 + rowsv[row0 + a, pl.ds(HD, LANES)] * pa
                    acc1 = acc1 + rowsv[row0 + a,
                                        pl.ds(HD + LANES, LANES)] * pa
                o_v[i, pl.ds(0, LANES)] = acc0
                o_v[i, pl.ds(LANES, LANES)] = acc1
                return carry

            lax.fori_loop(0, CH, item_body, 0)

        def half(c, cur, nxt):
            @pl.when(c + 1 < NP)
            def _():
                wait_in(c + 1, *nxt[:3])
                issue_gather(nxt[2], nxt[3])
            wait_gather(cur[2], cur[3])
            compute(cur[0], cur[1], cur[3])
            g0 = wid * IPW + c * CH
            pltpu.sync_copy(o_v, o_hbm.at[pl.ds(g0, CH)])
            pltpu.sync_copy(p_v, p_hbm.at[pl.ds(g0, CH)])

            @pl.when(c + 2 < NP)
            def _():
                issue_in(c + 2, *cur[:3])

        issue_in(0, *bufs[0][:3])
        wait_in(0, *bufs[0][:3])
        issue_gather(bufs[0][2], bufs[0][3])
        issue_in(1, *bufs[1][:3])

        def pair_body(p, carry):
            half(2 * p, bufs[0], bufs[1])
            half(2 * p + 1, bufs[1], bufs[0])
            return carry

        lax.fori_loop(0, NP // 2, pair_body, 0)

    return attn_core


_ATTN_CORE = _make_attn_core()


def _init_coords():
    gx, gy = jnp.meshgrid(jnp.arange(W, dtype=jnp.float32),
                          jnp.arange(H, dtype=jnp.float32), indexing='xy')
    return jnp.stack([gx, gy], axis=-1)[None]


def _layernorm(x, w, b, eps=1e-5):
    m = jnp.mean(x, -1, keepdims=True)
    v = jnp.var(x, -1, keepdims=True)
    return (x - m) / jnp.sqrt(v + eps) * w + b


def _window_meta(max_offset):
    ox = jnp.clip(max_offset[..., 0:1], float(R), W - 1 - R - 0.001)
    oy = jnp.clip(max_offset[..., 1:2], float(R), H - 1 - R - 0.001)
    mo = jnp.concatenate([ox, oy], -1)
    mf = jnp.floor(mo)
    f = mo - mf
    fx, fy = f[..., 0], f[..., 1]
    bw = jnp.stack([(1 - fx) * (1 - fy), fx * (1 - fy),
                    (1 - fx) * fy, fx * fy], -1)
    m_id = mf.astype(jnp.int32)
    base = m_id[..., 1] * W + m_id[..., 0]
    dy, dx = jnp.meshgrid(jnp.arange(-R, R + 2), jnp.arange(-R, R + 2),
                          indexing='ij')
    off = (dy * W + dx).reshape(-1).astype(jnp.int32)
    pos = jnp.clip(base[..., None] + off, 0, N - 1)
    h_arr = jnp.arange(NH, dtype=jnp.int32)[None, None, :, None]
    b_arr = jnp.arange(B, dtype=jnp.int32)[:, None, None, None]
    rows = (pos + b_arr * N) * NH + h_arr
    return rows.reshape(M * A), bw.reshape(M * 4)


def _attn_sc(q, k, v, max_offset):
    idx_sc, bw_sc = _window_meta(max_offset)
    q_sc = q.reshape(M, HD)
    kv_sc = jnp.concatenate([k, v], -1).reshape(M, 2 * HD)
    o, p = _ATTN_CORE(q_sc, kv_sc, idx_sc, bw_sc)
    return o.reshape(B, N, AD), p.reshape(B, N, NH, A)


def _bilinear_sample(img, coords):
    b, h, w, c = img.shape
    x = jnp.clip(coords[..., 0], 0.0, w - 1.0)
    y = jnp.clip(coords[..., 1], 0.0, h - 1.0)
    x0 = jnp.clip(jnp.floor(x).astype(jnp.int32), 0, w - 2)
    y0 = jnp.clip(jnp.floor(y).astype(jnp.int32), 0, h - 2)
    wx = (x - x0)[..., None]
    wy = (y - y0)[..., None]
    flat = img.reshape(b, h * w, c)

    def g(yy, xx):
        i = (yy * w + xx).reshape(b, h * w)
        return jnp.take_along_axis(flat, i[..., None], axis=1).reshape(b, h, w, c)

    v00 = g(y0, x0)
    v01 = g(y0, x0 + 1)
    v10 = g(y0 + 1, x0)
    v11 = g(y0 + 1, x0 + 1)
    return (v00 * (1 - wx) * (1 - wy) + v01 * wx * (1 - wy)
            + v10 * (1 - wx) * wy + v11 * wx * wy)


def _consistency_mask(field, a_thr=2.0):
    offset = field + _init_coords()
    half = B // 2
    field_tgt = jnp.concatenate([field[half:], field[:half]], axis=0)
    sampled = _bilinear_sample(field_tgt, offset)
    diff = jnp.abs(field + sampled).sum(-1, keepdims=True)
    return (diff < a_thr).astype(field.dtype)


def kernel(x, self_rpos, field, sa_qw, sa_kw, sa_vw, sa_projw, ca_qw, ca_kw,
           ca_vw, ca_gw, ca_projw, n0w, n0b, n1w, n1b, n2w, n2b, fc1w, fc1b,
           dww, dwb, fc2w, fc2b, field_scale):
    noc_mask = _consistency_mask(lax.stop_gradient(field))
    xx = jnp.concatenate([x, field * field_scale, self_rpos], -1)
    coords0 = jnp.tile(_init_coords(), (1, 1, 1, NH))
    self_offset = (self_rpos + coords0).reshape(B, N, NH, 2)

    xs = _layernorm(xx, n0w, n0b).reshape(B, N, DIM_S)
    xs = jnp.concatenate([xs, noc_mask.reshape(B, N, 1)], -1)
    q = (xs @ (sa_qw * SCALE)).reshape(B, N, NH, HD)
    k = (xs @ sa_kw).reshape(B, N, NH, HD)
    v = (xs @ sa_vw).reshape(B, N, NH, HD)
    out, _ = _attn_sc(q, k, v, self_offset)
    out = (out @ sa_projw).reshape(B, H, W, DIM_S)
    xx = xx + out

    self_rpos_out = xx[..., -(NH * 2):]
    xx = xx[..., :-(NH * 2)]
    xx = xx.at[..., -1].set(0.0)
    field1 = xx[..., -FD:] / field_scale
    offset = (jnp.tile(field1, (1, 1, 1, NH)) + coords0).reshape(B, N, NH, 2)

    xc = _layernorm(xx, n1w, n1b).reshape(B, N, DIM_C)
    half = B // 2
    tgt = jnp.concatenate([xc[half:], xc[:half]], axis=0)
    g = jax.nn.silu(xc @ ca_gw)
    q = (xc @ (ca_qw * SCALE)).reshape(B, N, NH, HD)
    k = (tgt @ ca_kw).reshape(B, N, NH, HD)
    v = (tgt @ ca_vw).reshape(B, N, NH, HD)
    out, probs = _attn_sc(q, k, v, offset)
    out = jnp.concatenate([g * out, probs.reshape(B, N, NH * A)], -1)
    out = (out @ ca_projw).reshape(B, H, W, DIM_C)
    xx = xx + out

    xx = xx.at[..., -1].set(0.0)
    field2 = xx[..., -FD:] / field_scale
    xx = xx[..., :-FD]

    hh = _layernorm(xx, n2w, n2b) @ fc1w + fc1b
    a, gg = jnp.split(hh, 2, axis=-1)
    a = lax.conv_general_dilated(
        a, dww, (1, 1), 'SAME',
        dimension_numbers=('NHWC', 'HWIO', 'NHWC'),
        feature_group_count=HID) + dwb
    xx = xx + (jax.nn.gelu(a) * gg) @ fc2w + fc2b
    return (xx, self_rpos_out, field2, field1, field2)


# speedup vs baseline: 69.1525x; 1.2189x over previous
import functools

import jax
import jax.numpy as jnp
from jax import lax
from jax.experimental import pallas as pl
from jax.experimental.pallas import tpu as pltpu
from jax.experimental.pallas import tpu_sc as plsc

B, H, W, DIM = 2, 128, 128, 96
NH, HD, R = 8, 32, 1
FD = 2
A = (2 * R + 2) * (2 * R + 2)
DIM_S = DIM + FD + NH * 2
EMB_S = DIM_S + 1
DIM_C = DIM + FD
AD = NH * HD
HID = DIM * 2
SCALE = HD ** (-0.5)
N = H * W
NC, NS, LANES = 2, 16, 16
NW = NC * NS
M = B * N * NH
CH = 32
NG = (CH * A) // 128
IPW = M // NW

_CORNERS = ((0, 0), (1, 0), (0, 1), (1, 1))


def _make_attn_core():
    mesh = plsc.VectorSubcoreMesh(
        core_axis_name="c", subcore_axis_name="s",
        num_cores=NC, num_subcores=NS)

    @functools.partial(
        pl.kernel,
        out_type=[jax.ShapeDtypeStruct((M, HD), jnp.float32),
                  jax.ShapeDtypeStruct((M, A), jnp.float32)],
        mesh=mesh,
        compiler_params=pltpu.CompilerParams(
            use_tc_tiling_on_sc=False, needs_layout_passes=False),
        scratch_types=[
            pltpu.VMEM((CH, HD), jnp.float32),
            pltpu.VMEM((CH, HD), jnp.float32),
            pltpu.VMEM((CH * 4 + LANES,), jnp.float32),
            pltpu.VMEM((CH * 4 + LANES,), jnp.float32),
            pltpu.VMEM((CH * A,), jnp.int32),
            pltpu.VMEM((CH * A,), jnp.int32),
            pltpu.VMEM((CH * A, 2 * HD), jnp.float32),
            pltpu.VMEM((CH * A, 2 * HD), jnp.float32),
            pltpu.VMEM((CH, HD), jnp.float32),
            pltpu.VMEM((CH, HD), jnp.float32),
            pltpu.VMEM((CH, A), jnp.float32),
            pltpu.VMEM((CH, A), jnp.float32),
            pltpu.SemaphoreType.DMA,
            pltpu.SemaphoreType.DMA,
            pltpu.SemaphoreType.DMA,
        ])
    def attn_core(q_hbm, kv_hbm, idx_hbm, bw_hbm, o_hbm, p_hbm,
                  q_0, q_1, bw_0, bw_1, idx_0, idx_1, rows_0, rows_1,
                  o_0, o_1, p_0, p_1, isem, gsem, osem):
        cid = lax.axis_index("c")
        sid = lax.axis_index("s")
        wid = cid * NS + sid
        NP = IPW // CH
        bufs = [(q_0, bw_0, idx_0, rows_0, o_0, p_0),
                (q_1, bw_1, idx_1, rows_1, o_1, p_1)]

        def in_copies(c, qv, bwv, idxv):
            g0 = wid * IPW + c * CH
            return [(q_hbm.at[pl.ds(g0, CH)], qv),
                    (bw_hbm.at[pl.ds(g0 * 4, CH * 4)],
                     bwv.at[pl.ds(0, CH * 4)]),
                    (idx_hbm.at[pl.ds(g0 * A, CH * A)], idxv)]

        def issue_in(c, qv, bwv, idxv):
            for s, d in in_copies(c, qv, bwv, idxv):
                pltpu.async_copy(s, d, isem)

        def wait_in(c, qv, bwv, idxv):
            for s, d in in_copies(c, qv, bwv, idxv):
                pltpu.make_async_copy(s, d, isem).wait()

        def issue_gather(idxv, rowsv):
            for j in range(NG):
                pltpu.async_copy(kv_hbm.at[idxv.at[pl.ds(j * 128, 128)]],
                                 rowsv.at[pl.ds(j * 128, 128)], gsem)

        def wait_gather(idxv, rowsv):
            for j in range(NG):
                pltpu.make_async_copy(
                    kv_hbm.at[idxv.at[pl.ds(j * 128, 128)]],
                    rowsv.at[pl.ds(j * 128, 128)], gsem).wait()

        def out_copies(c, ob, pb):
            g0 = wid * IPW + c * CH
            return [(ob, o_hbm.at[pl.ds(g0, CH)]),
                    (pb, p_hbm.at[pl.ds(g0, CH)])]

        def issue_out(c, ob, pb):
            for s, d in out_copies(c, ob, pb):
                pltpu.async_copy(s, d, osem)

        def wait_out(c, ob, pb):
            for s, d in out_copies(c, ob, pb):
                pltpu.make_async_copy(s, d, osem).wait()

        def compute(qv, bwv, rowsv, ob, pb):
            def one_item(i):
                lane = lax.iota(jnp.int32, 16)
                dyv = lane // 4
                dxv = lane - dyv * 4
                msks = [((dyv >= dyc) & (dyv <= dyc + 2)
                         & (dxv >= dxc) & (dxv <= dxc + 2))
                        for (dxc, dyc) in _CORNERS]
                row0 = i * A
                qa = qv[i, pl.ds(0, LANES)]
                qb = qv[i, pl.ds(LANES, LANES)]
                acc = jnp.zeros((LANES,), jnp.float32)
                for a in range(A):
                    k0 = rowsv[row0 + a, pl.ds(0, LANES)]
                    k1 = rowsv[row0 + a, pl.ds(LANES, LANES)]
                    s = jnp.sum(k0 * qa + k1 * qb)
                    acc = jnp.where(lane == a, s, acc)
                ex = jnp.exp(acc)
                bwi = bwv[pl.ds(i * 4, LANES)]
                pv = jnp.zeros((LANES,), jnp.float32)
                for ci in range(4):
                    e = jnp.where(msks[ci], ex, 0.0)
                    pv = pv + (e * bwi[ci]) / jnp.sum(e)
                pb[i] = pv
                acc0 = jnp.zeros((LANES,), jnp.float32)
                acc1 = jnp.zeros((LANES,), jnp.float32)
                for a in range(A):
                    pa = pv[a]
                    acc0 = acc0 + rowsv[row0 + a, pl.ds(HD, LANES)] * pa
                    acc1 = acc1 + rowsv[row0 + a,
                                        pl.ds(HD + LANES, LANES)] * pa
                ob[i, pl.ds(0, LANES)] = acc0
                ob[i, pl.ds(LANES, LANES)] = acc1

            def item_body(i2, carry):
                one_item(2 * i2)
                one_item(2 * i2 + 1)
                return carry

            lax.fori_loop(0, CH // 2, item_body, 0)

        def half(c, cur, nxt):
            @pl.when(c + 1 < NP)
            def _():
                wait_in(c + 1, *nxt[:3])
                issue_gather(nxt[2], nxt[3])
            wait_gather(cur[2], cur[3])

            @pl.when(c >= 2)
            def _():
                wait_out(c - 2, cur[4], cur[5])

            compute(cur[0], cur[1], cur[3], cur[4], cur[5])
            issue_out(c, cur[4], cur[5])

            @pl.when(c + 2 < NP)
            def _():
                issue_in(c + 2, *cur[:3])

        issue_in(0, *bufs[0][:3])
        wait_in(0, *bufs[0][:3])
        issue_gather(bufs[0][2], bufs[0][3])
        issue_in(1, *bufs[1][:3])

        def pair_body(p, carry):
            half(2 * p, bufs[0], bufs[1])
            half(2 * p + 1, bufs[1], bufs[0])
            return carry

        lax.fori_loop(0, NP // 2, pair_body, 0)
        wait_out(NP - 2, bufs[0][4], bufs[0][5])
        wait_out(NP - 1, bufs[1][4], bufs[1][5])

    return attn_core


_ATTN_CORE = _make_attn_core()


def _init_coords():
    gx, gy = jnp.meshgrid(jnp.arange(W, dtype=jnp.float32),
                          jnp.arange(H, dtype=jnp.float32), indexing='xy')
    return jnp.stack([gx, gy], axis=-1)[None]


def _layernorm(x, w, b, eps=1e-5):
    m = jnp.mean(x, -1, keepdims=True)
    v = jnp.var(x, -1, keepdims=True)
    return (x - m) / jnp.sqrt(v + eps) * w + b


def _window_meta(max_offset):
    ox = jnp.clip(max_offset[..., 0:1], float(R), W - 1 - R - 0.001)
    oy = jnp.clip(max_offset[..., 1:2], float(R), H - 1 - R - 0.001)
    mo = jnp.concatenate([ox, oy], -1)
    mf = jnp.floor(mo)
    f = mo - mf
    fx, fy = f[..., 0], f[..., 1]
    bw = jnp.stack([(1 - fx) * (1 - fy), fx * (1 - fy),
                    (1 - fx) * fy, fx * fy], -1)
    m_id = mf.astype(jnp.int32)
    base = m_id[..., 1] * W + m_id[..., 0]
    dy, dx = jnp.meshgrid(jnp.arange(-R, R + 2), jnp.arange(-R, R + 2),
                          indexing='ij')
    off = (dy * W + dx).reshape(-1).astype(jnp.int32)
    pos = jnp.clip(base[..., None] + off, 0, N - 1)
    h_arr = jnp.arange(NH, dtype=jnp.int32)[None, None, :, None]
    b_arr = jnp.arange(B, dtype=jnp.int32)[:, None, None, None]
    rows = (pos + b_arr * N) * NH + h_arr
    return rows.reshape(M * A), bw.reshape(M * 4)


def _attn_sc(q, k, v, max_offset):
    idx_sc, bw_sc = _window_meta(max_offset)
    q_sc = q.reshape(M, HD)
    kv_sc = jnp.concatenate([k, v], -1).reshape(M, 2 * HD)
    o, p = _ATTN_CORE(q_sc, kv_sc, idx_sc, bw_sc)
    return o.reshape(B, N, AD), p.reshape(B, N, NH, A)


def _bilinear_sample(img, coords):
    b, h, w, c = img.shape
    x = jnp.clip(coords[..., 0], 0.0, w - 1.0)
    y = jnp.clip(coords[..., 1], 0.0, h - 1.0)
    x0 = jnp.clip(jnp.floor(x).astype(jnp.int32), 0, w - 2)
    y0 = jnp.clip(jnp.floor(y).astype(jnp.int32), 0, h - 2)
    wx = (x - x0)[..., None]
    wy = (y - y0)[..., None]
    flat = img.reshape(b, h * w, c)

    def g(yy, xx):
        i = (yy * w + xx).reshape(b, h * w)
        return jnp.take_along_axis(flat, i[..., None], axis=1).reshape(b, h, w, c)

    v00 = g(y0, x0)
    v01 = g(y0, x0 + 1)
    v10 = g(y0 + 1, x0)
    v11 = g(y0 + 1, x0 + 1)
    return (v00 * (1 - wx) * (1 - wy) + v01 * wx * (1 - wy)
            + v10 * (1 - wx) * wy + v11 * wx * wy)


def _consistency_mask(field, a_thr=2.0):
    offset = field + _init_coords()
    half = B // 2
    field_tgt = jnp.concatenate([field[half:], field[:half]], axis=0)
    sampled = _bilinear_sample(field_tgt, offset)
    diff = jnp.abs(field + sampled).sum(-1, keepdims=True)
    return (diff < a_thr).astype(field.dtype)


def kernel(x, self_rpos, field, sa_qw, sa_kw, sa_vw, sa_projw, ca_qw, ca_kw,
           ca_vw, ca_gw, ca_projw, n0w, n0b, n1w, n1b, n2w, n2b, fc1w, fc1b,
           dww, dwb, fc2w, fc2b, field_scale):
    noc_mask = _consistency_mask(lax.stop_gradient(field))
    xx = jnp.concatenate([x, field * field_scale, self_rpos], -1)
    coords0 = jnp.tile(_init_coords(), (1, 1, 1, NH))
    self_offset = (self_rpos + coords0).reshape(B, N, NH, 2)

    xs = _layernorm(xx, n0w, n0b).reshape(B, N, DIM_S)
    xs = jnp.concatenate([xs, noc_mask.reshape(B, N, 1)], -1)
    q = (xs @ (sa_qw * SCALE)).reshape(B, N, NH, HD)
    k = (xs @ sa_kw).reshape(B, N, NH, HD)
    v = (xs @ sa_vw).reshape(B, N, NH, HD)
    out, _ = _attn_sc(q, k, v, self_offset)
    out = (out @ sa_projw).reshape(B, H, W, DIM_S)
    xx = xx + out

    self_rpos_out = xx[..., -(NH * 2):]
    xx = xx[..., :-(NH * 2)]
    xx = xx.at[..., -1].set(0.0)
    field1 = xx[..., -FD:] / field_scale
    offset = (jnp.tile(field1, (1, 1, 1, NH)) + coords0).reshape(B, N, NH, 2)

    xc = _layernorm(xx, n1w, n1b).reshape(B, N, DIM_C)
    half = B // 2
    tgt = jnp.concatenate([xc[half:], xc[:half]], axis=0)
    g = jax.nn.silu(xc @ ca_gw)
    q = (xc @ (ca_qw * SCALE)).reshape(B, N, NH, HD)
    k = (tgt @ ca_kw).reshape(B, N, NH, HD)
    v = (tgt @ ca_vw).reshape(B, N, NH, HD)
    out, probs = _attn_sc(q, k, v, offset)
    out = jnp.concatenate([g * out, probs.reshape(B, N, NH * A)], -1)
    out = (out @ ca_projw).reshape(B, H, W, DIM_C)
    xx = xx + out

    xx = xx.at[..., -1].set(0.0)
    field2 = xx[..., -FD:] / field_scale
    xx = xx[..., :-FD]

    hh = _layernorm(xx, n2w, n2b) @ fc1w + fc1b
    a, gg = jnp.split(hh, 2, axis=-1)
    a = lax.conv_general_dilated(
        a, dww, (1, 1), 'SAME',
        dimension_numbers=('NHWC', 'HWIO', 'NHWC'),
        feature_group_count=HID) + dwb
    xx = xx + (jax.nn.gelu(a) * gg) @ fc2w + fc2b
    return (xx, self_rpos_out, field2, field1, field2)
